# baseline (speedup 1.0000x reference)
import functools

import jax
import jax.numpy as jnp
from jax import lax
from jax.experimental import pallas as pl
from jax.experimental.pallas import tpu as pltpu
from jax.experimental.pallas import tpu_sc as plsc

N = 10000
E = 320000
IN = 128
HID = 64
HID2 = 32
HEADS = 4

NC, NS = 2, 16
NW = NC * NS
CH = 128
NP = 10240
PE = 327680
PCHUNK = PE // CH
FULL = PCHUNK // NW
RPS = NP // NS

WA = 80
WB_S = 80
WB_D = 16
WB_O = 72
WC = 32


def _mesh():
    return plsc.VectorSubcoreMesh(core_axis_name="c", subcore_axis_name="s")


def _stage_indices(src_h, dst_h, src_v, dst_v, w):
    pltpu.sync_copy(src_h.at[pl.ds(w * FULL, FULL)], src_v)
    pltpu.sync_copy(dst_h.at[pl.ds(w * FULL, FULL)], dst_v)


def _seg_sum(table, src2d, dst2d, zeros, width):
    @functools.partial(
        pl.kernel,
        out_type=jax.ShapeDtypeStruct((NC, NP, width), jnp.float32),
        mesh=_mesh(),
        compiler_params=pltpu.CompilerParams(use_tc_tiling_on_sc=False, needs_layout_passes=False),
        scratch_types=[
            pltpu.VMEM((FULL, CH), jnp.int32),
            pltpu.VMEM((FULL, CH), jnp.int32),
            pltpu.VMEM((4, CH, width), jnp.float32),
            pltpu.VMEM_SHARED((NP, width), jnp.float32),
            [pltpu.SemaphoreType.DMA] * 4,
        ],
    )
    def k(table_h, src_h, dst_h, zeros_h, out_h, src_v, dst_v, rows_v, acc,
          gsems):
        cid = lax.axis_index("c")
        sid = lax.axis_index("s")
        w = sid * NC + cid
        pltpu.sync_copy(zeros_h.at[pl.ds(sid * RPS, RPS)],
                        acc.at[pl.ds(sid * RPS, RPS)])
        _stage_indices(src_h, dst_h, src_v, dst_v, w)
        plsc.subcore_barrier()

        for b in range(4):
            pltpu.async_copy(table_h.at[src_v.at[b]], rows_v.at[b], gsems[b])

        def gbody(q, carry):
            for b in range(4):
                j = q * 4 + b
                pltpu.make_async_copy(table_h.at[src_v.at[j]],
                                      rows_v.at[b], gsems[b]).wait()
                pltpu.sync_copy(rows_v.at[b], acc.at[dst_v.at[j]], add=True)
                nj = j + 4

                @pl.when(nj < FULL)
                def _():
                    pltpu.async_copy(table_h.at[src_v.at[nj]],
                                     rows_v.at[b], gsems[b])
            return carry

        lax.fori_loop(0, FULL // 4, gbody, 0)
        plsc.subcore_barrier()
        pltpu.sync_copy(acc.at[pl.ds(sid * RPS, RPS)],
                        out_h.at[cid, pl.ds(sid * RPS, RPS)])

    return k(table, src2d, dst2d, zeros)


def _gat_edge(t_src, t_dst, src2d, dst2d, zeros):
    @functools.partial(
        pl.kernel,
        out_type=jax.ShapeDtypeStruct((NC, 2, NP, WB_O), jnp.float32),
        mesh=_mesh(),
        compiler_params=pltpu.CompilerParams(use_tc_tiling_on_sc=False, needs_layout_passes=False),
        scratch_types=[
            pltpu.VMEM((FULL, CH), jnp.int32),
            pltpu.VMEM((FULL, CH), jnp.int32),
            pltpu.VMEM((4, CH, WB_S), jnp.float32),
            pltpu.VMEM((4, CH, WB_D), jnp.float32),
            pltpu.VMEM((1, CH, WB_O), jnp.float32),
            pltpu.VMEM_SHARED((NP, WB_O), jnp.float32),
            [pltpu.SemaphoreType.DMA] * 4,
            [pltpu.SemaphoreType.DMA] * 4,
        ],
    )
    def k(ts_h, td_h, src_h, dst_h, zeros_h, out_h,
          src_v, dst_v, srows, drows, orows, acc, sem_s, sem_d):
        cid = lax.axis_index("c")
        sid = lax.axis_index("s")
        w = sid * NC + cid
        _stage_indices(src_h, dst_h, src_v, dst_v, w)

        zero16 = jnp.zeros((16,), jnp.float32)

        def zbody(rz, carry):
            orows[0, rz, pl.ds(HID - 8, 16)] = zero16
            return carry

        lax.fori_loop(0, CH, zbody, 0)

        for p in range(2):
            pltpu.sync_copy(zeros_h.at[pl.ds(sid * RPS, RPS)],
                            acc.at[pl.ds(sid * RPS, RPS)])
            plsc.subcore_barrier()
            cals = 64 + 2 * cid + p
            cald = 2 * cid + p

            for b in range(4):
                pltpu.async_copy(ts_h.at[src_v.at[b]], srows.at[b], sem_s[b])
                pltpu.async_copy(td_h.at[dst_v.at[b]], drows.at[b], sem_d[b])

            def pbody(q, carry):
                for b in range(4):
                    j = q * 4 + b
                    o = 0
                    pltpu.make_async_copy(ts_h.at[src_v.at[j]],
                                          srows.at[b], sem_s[b]).wait()
                    pltpu.make_async_copy(td_h.at[dst_v.at[j]],
                                          drows.at[b], sem_d[b]).wait()

                    def group(g, gc):
                        ie = g * 16 + lax.iota(jnp.int32, 16)
                        als = plsc.load_gather(
                            srows.at[b], [ie, jnp.full((16,), cals, jnp.int32)])
                        ald = plsc.load_gather(
                            drows.at[b], [ie, jnp.full((16,), cald, jnp.int32)])
                        cc = plsc.load_gather(
                            drows.at[b],
                            [ie, jnp.full((16,), 4 + cald, jnp.int32)])
                        e = als + ald
                        e = jnp.where(e > 0.0, e, 0.2 * e)
                        ex = jnp.exp(e - cc)
                        plsc.store_scatter(
                            orows.at[o], [ie, jnp.full((16,), HID, jnp.int32)],
                            ex)
                        for jj in range(16):
                            rr = g * 16 + jj
                            eb = jnp.full((16,), ex[jj], jnp.float32)
                            for kk in range(4):
                                v = srows[b, rr, pl.ds(kk * 16, 16)]
                                orows[o, rr, pl.ds(kk * 16, 16)] = v * eb
                        return gc

                    lax.fori_loop(0, CH // 16, group, 0)
                    pltpu.sync_copy(orows.at[o], acc.at[dst_v.at[j]], add=True)
                    nj = j + 4

                    @pl.when(nj < FULL)
                    def _():
                        pltpu.async_copy(ts_h.at[src_v.at[nj]],
                                         srows.at[b], sem_s[b])
                        pltpu.async_copy(td_h.at[dst_v.at[nj]],
                                         drows.at[b], sem_d[b])
                return carry

            lax.fori_loop(0, FULL // 4, pbody, 0)
            plsc.subcore_barrier()
            pltpu.sync_copy(acc.at[pl.ds(sid * RPS, RPS)],
                            out_h.at[cid, p, pl.ds(sid * RPS, RPS)])

    return k(t_src, t_dst, src2d, dst2d, zeros)


def _dotg(a, b):
    return lax.dot_general(a, b, (((1,), (1,)), ((), ())),
                           preferred_element_type=jnp.float32)


def _tc1(x, W_l1, b_l1, W_r1, gamma1, beta1, rm1, rv1, Ws, bs):
    def body(x_r, wl_r, bl_r, wr_r, g_r, be_r, rm_r, rv_r, ws_r, bs_r,
             ta_r, xr_r, idn_r):
        s1 = g_r[...] * lax.rsqrt(rv_r[...] + 1e-5)
        t1 = be_r[...] - rm_r[...] * s1
        xv = x_r[...]
        ta_r[0:N, 0:HID] = _dotg(xv, wl_r[...]) * s1
        ta_r[0:N, HID:HID + 1] = jnp.ones((N, 1), jnp.float32)
        ta_r[0:N, HID + 1:WA] = jnp.zeros((N, WA - HID - 1), jnp.float32)
        ta_r[N:NP, :] = jnp.zeros((NP - N, WA), jnp.float32)
        xr_r[...] = _dotg(xv, wr_r[...]) * s1 + (bl_r[...] * s1 + t1)
        idn_r[...] = _dotg(xv, ws_r[...]) + bs_r[...]

    return pl.pallas_call(
        body,
        out_shape=(
            jax.ShapeDtypeStruct((NP, WA), jnp.float32),
            jax.ShapeDtypeStruct((N, HID), jnp.float32),
            jax.ShapeDtypeStruct((N, HID2), jnp.float32),
        ),
    )(x, W_l1, b_l1, W_r1, gamma1, beta1, rm1, rv1, Ws, bs)


def _tc2(P, xr, Wg, a_src, a_dst):
    def body(p_r, xr_r, wg_r, as_r, ad_r, ts_r, td_r, cnt_r):
        S = p_r[0][0:N] + p_r[1][0:N]
        cnt = jnp.maximum(S[:, HID:HID + 1], 1.0)
        h1 = jnp.maximum(S[:, 0:HID] / cnt + xr_r[...], 0.0)
        wg = wg_r[...]
        def _proj(avec, kk):
            return lax.dot_general(avec, wg[kk * HID:(kk + 1) * HID, :],
                                   (((1,), (0,)), ((), ())),
                                   preferred_element_type=jnp.float32)
        rows = []
        for kk in range(HEADS):
            rows.append(_proj(as_r[kk:kk + 1, :], kk))
        for kk in range(HEADS):
            rows.append(_proj(ad_r[kk:kk + 1, :], kk))
        M = jnp.concatenate(rows, axis=0)
        aa = _dotg(h1, M)
        als = aa[:, 0:HEADS]
        ald = aa[:, HEADS:2 * HEADS]
        gmax = jnp.max(als, axis=0, keepdims=True)
        z = gmax + ald
        cc = jnp.where(z > 0.0, z, 0.2 * z)
        ts_r[0:N, 0:HID] = h1
        ts_r[0:N, HID:HID + HEADS] = als
        ts_r[0:N, HID + HEADS:WB_S] = jnp.zeros((N, WB_S - HID - HEADS), jnp.float32)
        ts_r[N:NP, :] = jnp.zeros((NP - N, WB_S), jnp.float32)
        td_r[0:N, 0:HEADS] = ald
        td_r[0:N, HEADS:2 * HEADS] = cc
        td_r[0:N, 2 * HEADS:WB_D] = jnp.zeros((N, WB_D - 2 * HEADS), jnp.float32)
        td_r[N:NP, :] = jnp.zeros((NP - N, WB_D), jnp.float32)
        cnt_r[...] = jnp.broadcast_to(cnt, (N, 8))

    return pl.pallas_call(
        body,
        out_shape=(
            jax.ShapeDtypeStruct((NP, WB_S), jnp.float32),
            jax.ShapeDtypeStruct((NP, WB_D), jnp.float32),
            jax.ShapeDtypeStruct((N, 8), jnp.float32),
        ),
    )(P, xr, Wg, a_src, a_dst)


def _tc3(P2, Wg, bg, gamma2, beta2, rm2, rv2,
         W_l3, b_l3, W_r3, gamma3, beta3, rm3, rv3):
    def body(p2_r, wg_r, bg_r, g2_r, b2_r, rm2_r, rv2_r,
             wl3_r, bl3_r, wr3_r, g3_r, b3_r, rm3_r, rv3_r, tb_r, h2r_r):
        wg = wg_r[...]
        gat = jnp.zeros((N, HID), jnp.float32)
        for kk in range(HEADS):
            ci, slot = kk // 2, kk % 2
            numer = p2_r[ci, slot][0:N, 0:HID]
            den = p2_r[ci, slot][0:N, HID:HID + 1]
            ok = _dotg(numer, wg[kk * HID:(kk + 1) * HID, :])
            gat = gat + jnp.where(den > 0.0, ok / jnp.maximum(den, 1e-30), 0.0)
        gat = gat * 0.25 + bg_r[...]
        s2 = g2_r[...] * lax.rsqrt(rv2_r[...] + 1e-5)
        t2 = b2_r[...] - rm2_r[...] * s2
        h2 = jnp.maximum(gat * s2 + t2, 0.0)
        s3 = g3_r[...] * lax.rsqrt(rv3_r[...] + 1e-5)
        t3 = b3_r[...] - rm3_r[...] * s3
        tb_r[0:N, :] = _dotg(h2, wl3_r[...]) * s3
        tb_r[N:NP, :] = jnp.zeros((NP - N, WC), jnp.float32)
        h2r_r[...] = _dotg(h2, wr3_r[...]) * s3 + (bl3_r[...] * s3 + t3)

    return pl.pallas_call(
        body,
        out_shape=(
            jax.ShapeDtypeStruct((NP, WC), jnp.float32),
            jax.ShapeDtypeStruct((N, HID2), jnp.float32),
        ),
    )(P2, Wg, bg, gamma2, beta2, rm2, rv2,
      W_l3, b_l3, W_r3, gamma3, beta3, rm3, rv3)


def _tc4(P3, cntv, h2r, idn, Wc1, bc1, Wc2, bc2):
    def body(p3_r, cnt_r, h2r_r, idn_r, wc1_r, bc1_r, wc2_r, bc2_r, out_r):
        S3 = p3_r[0][0:N] + p3_r[1][0:N]
        cnt = cnt_r[:, 0:1]
        h3 = jnp.maximum(S3 / cnt + h2r_r[...], 0.0)
        emb = h3 + idn_r[...]
        z = jnp.maximum(_dotg(emb, wc1_r[...]) + bc1_r[...], 0.0)
        l = _dotg(z, wc2_r[...]) + bc2_r[...]
        m = jnp.max(l, axis=1, keepdims=True)
        lse = m + jnp.log(jnp.sum(jnp.exp(l - m), axis=1, keepdims=True))
        out_r[...] = l - lse

    return pl.pallas_call(
        body,
        out_shape=jax.ShapeDtypeStruct((N, 2), jnp.float32),
    )(P3, cntv, h2r, idn, Wc1, bc1, Wc2, bc2)


def kernel(x, edge_index, W_l1, b_l1, W_r1, gamma1, beta1, rm1, rv1,
           Wg, a_src, a_dst, bg, gamma2, beta2, rm2, rv2,
           W_l3, b_l3, W_r3, gamma3, beta3, rm3, rv3,
           Ws, bs, Wc1, bc1, Wc2, bc2):
    pad_row = N + (jnp.arange(PE - E, dtype=jnp.int32) % (NP - N))
    src2d = jnp.concatenate([edge_index[0], pad_row]).reshape(PCHUNK, CH)
    dst2d = jnp.concatenate([edge_index[1], pad_row]).reshape(PCHUNK, CH)
    r = lambda v: v.reshape(1, -1)

    tA, xr, idn = _tc1(x, W_l1, r(b_l1), W_r1, r(gamma1), r(beta1),
                       r(rm1), r(rv1), Ws, r(bs))
    P = _seg_sum(tA, src2d, dst2d, jnp.zeros((NP, WA), jnp.float32), WA)
    tS, tD, cntv = _tc2(P, xr, Wg, a_src, a_dst)
    P2 = _gat_edge(tS, tD, src2d, dst2d, jnp.zeros((NP, WB_O), jnp.float32))
    tB, h2r = _tc3(P2, Wg, r(bg), r(gamma2), r(beta2), r(rm2), r(rv2),
                   W_l3, r(b_l3), W_r3, r(gamma3), r(beta3), r(rm3), r(rv3))
    P3 = _seg_sum(tB, src2d, dst2d, jnp.zeros((NP, WC), jnp.float32), WC)
    return _tc4(P3, cntv, h2r, idn, Wc1, r(bc1), Wc2, r(bc2))


# speedup vs baseline: 1.0903x; 1.0903x over previous
import functools

import jax
import jax.numpy as jnp
from jax import lax
from jax.experimental import pallas as pl
from jax.experimental.pallas import tpu as pltpu
from jax.experimental.pallas import tpu_sc as plsc

N = 10000
E = 320000
IN = 128
HID = 64
HID2 = 32
HEADS = 4

NC, NS = 2, 16
NW = NC * NS
CH = 128
NP = 10240
PE = 327680
PCHUNK = PE // CH
FULL = PCHUNK // NW
RPS = NP // NS

WA = 80
WB_S = 80
WB_D = 16
WB_O = 72
WC = 32


def _mesh():
    return plsc.VectorSubcoreMesh(core_axis_name="c", subcore_axis_name="s")


def _stage_indices(src_h, dst_h, src_v, dst_v, w):
    pltpu.sync_copy(src_h.at[pl.ds(w * FULL, FULL)], src_v)
    pltpu.sync_copy(dst_h.at[pl.ds(w * FULL, FULL)], dst_v)


def _seg_sum(table, src2d, dst2d, zeros, width):
    @functools.partial(
        pl.kernel,
        out_type=jax.ShapeDtypeStruct((NC, NP, width), jnp.float32),
        mesh=_mesh(),
        compiler_params=pltpu.CompilerParams(use_tc_tiling_on_sc=False, needs_layout_passes=False),
        scratch_types=[
            pltpu.VMEM((FULL, CH), jnp.int32),
            pltpu.VMEM((FULL, CH), jnp.int32),
            pltpu.VMEM((4, CH, width), jnp.float32),
            pltpu.VMEM_SHARED((NP, width), jnp.float32),
            [pltpu.SemaphoreType.DMA] * 4,
        ],
    )
    def k(table_h, src_h, dst_h, zeros_h, out_h, src_v, dst_v, rows_v, acc,
          gsems):
        cid = lax.axis_index("c")
        sid = lax.axis_index("s")
        w = sid * NC + cid
        pltpu.sync_copy(zeros_h.at[pl.ds(sid * RPS, RPS)],
                        acc.at[pl.ds(sid * RPS, RPS)])
        _stage_indices(src_h, dst_h, src_v, dst_v, w)
        plsc.subcore_barrier()

        for b in range(4):
            pltpu.async_copy(table_h.at[src_v.at[b]], rows_v.at[b], gsems[b])

        def gbody(q, carry):
            for b in range(4):
                j = q * 4 + b
                pltpu.make_async_copy(table_h.at[src_v.at[j]],
                                      rows_v.at[b], gsems[b]).wait()
                pltpu.sync_copy(rows_v.at[b], acc.at[dst_v.at[j]], add=True)
                nj = j + 4

                @pl.when(nj < FULL)
                def _():
                    pltpu.async_copy(table_h.at[src_v.at[nj]],
                                     rows_v.at[b], gsems[b])
            return carry

        lax.fori_loop(0, FULL // 4, gbody, 0)
        plsc.subcore_barrier()
        pltpu.sync_copy(acc.at[pl.ds(sid * RPS, RPS)],
                        out_h.at[cid, pl.ds(sid * RPS, RPS)])

    return k(table, src2d, dst2d, zeros)


def _gat_edge(t_src, t_dst, src2d, dst2d, zeros):
    @functools.partial(
        pl.kernel,
        out_type=jax.ShapeDtypeStruct((NC, 2, NP, WB_O), jnp.float32),
        mesh=_mesh(),
        compiler_params=pltpu.CompilerParams(use_tc_tiling_on_sc=False, needs_layout_passes=False),
        scratch_types=[
            pltpu.VMEM((FULL, CH), jnp.int32),
            pltpu.VMEM((FULL, CH), jnp.int32),
            pltpu.VMEM((2, CH, WB_S), jnp.float32),
            pltpu.VMEM((2, CH, WB_D), jnp.float32),
            pltpu.VMEM((2, CH, WB_O), jnp.float32),
            pltpu.VMEM_SHARED((NP, WB_O), jnp.float32),
            [pltpu.SemaphoreType.DMA] * 2,
            [pltpu.SemaphoreType.DMA] * 2,
            [pltpu.SemaphoreType.DMA] * 2,
        ],
    )
    def k(ts_h, td_h, src_h, dst_h, zeros_h, out_h,
          src_v, dst_v, srows, drows, orows, acc, sem_s, sem_d, sem_o):
        cid = lax.axis_index("c")
        sid = lax.axis_index("s")
        w = sid * NC + cid
        _stage_indices(src_h, dst_h, src_v, dst_v, w)

        zero16 = jnp.zeros((16,), jnp.float32)

        def zbody(rz, carry):
            orows[0, rz, pl.ds(HID - 8, 16)] = zero16
            orows[1, rz, pl.ds(HID - 8, 16)] = zero16
            return carry

        lax.fori_loop(0, CH, zbody, 0)

        for p in range(2):
            pltpu.sync_copy(zeros_h.at[pl.ds(sid * RPS, RPS)],
                            acc.at[pl.ds(sid * RPS, RPS)])
            plsc.subcore_barrier()
            cals = 64 + 2 * cid + p
            cald = 2 * cid + p

            for b in range(2):
                pltpu.async_copy(ts_h.at[src_v.at[b]], srows.at[b], sem_s[b])
                pltpu.async_copy(td_h.at[dst_v.at[b]], drows.at[b], sem_d[b])

            def pbody(q, carry):
                for b in range(2):
                    j = q * 2 + b
                    o = b
                    pltpu.make_async_copy(ts_h.at[src_v.at[j]],
                                          srows.at[b], sem_s[b]).wait()
                    pltpu.make_async_copy(td_h.at[dst_v.at[j]],
                                          drows.at[b], sem_d[b]).wait()

                    @pl.when(j >= 2)
                    def _():
                        pltpu.make_async_copy(orows.at[o],
                                              acc.at[dst_v.at[j]],
                                              sem_o[o]).wait()

                    def group(g, gc):
                        ie = g * 16 + lax.iota(jnp.int32, 16)
                        als = plsc.load_gather(
                            srows.at[b], [ie, jnp.full((16,), cals, jnp.int32)])
                        ald = plsc.load_gather(
                            drows.at[b], [ie, jnp.full((16,), cald, jnp.int32)])
                        cc = plsc.load_gather(
                            drows.at[b],
                            [ie, jnp.full((16,), 4 + cald, jnp.int32)])
                        e = als + ald
                        e = jnp.where(e > 0.0, e, 0.2 * e)
                        ex = jnp.exp(e - cc)
                        plsc.store_scatter(
                            orows.at[o], [ie, jnp.full((16,), HID, jnp.int32)],
                            ex)
                        for jj in range(16):
                            rr = g * 16 + jj
                            eb = jnp.full((16,), ex[jj], jnp.float32)
                            for kk in range(4):
                                v = srows[b, rr, pl.ds(kk * 16, 16)]
                                orows[o, rr, pl.ds(kk * 16, 16)] = v * eb
                        return gc

                    lax.fori_loop(0, CH // 16, group, 0)
                    pltpu.async_copy(orows.at[o], acc.at[dst_v.at[j]],
                                     sem_o[o], add=True)
                    nj = j + 2

                    @pl.when(nj < FULL)
                    def _():
                        pltpu.async_copy(ts_h.at[src_v.at[nj]],
                                         srows.at[b], sem_s[b])
                        pltpu.async_copy(td_h.at[dst_v.at[nj]],
                                         drows.at[b], sem_d[b])
                return carry

            lax.fori_loop(0, FULL // 2, pbody, 0)
            for o in range(2):
                pltpu.make_async_copy(orows.at[o], acc.at[dst_v.at[0]],
                                      sem_o[o]).wait()
            plsc.subcore_barrier()
            pltpu.sync_copy(acc.at[pl.ds(sid * RPS, RPS)],
                            out_h.at[cid, p, pl.ds(sid * RPS, RPS)])

    return k(t_src, t_dst, src2d, dst2d, zeros)


def _dotg(a, b):
    return lax.dot_general(a, b, (((1,), (1,)), ((), ())),
                           preferred_element_type=jnp.float32)


def _tc1(x, W_l1, b_l1, W_r1, gamma1, beta1, rm1, rv1, Ws, bs):
    def body(x_r, wl_r, bl_r, wr_r, g_r, be_r, rm_r, rv_r, ws_r, bs_r,
             ta_r, xr_r, idn_r):
        s1 = g_r[...] * lax.rsqrt(rv_r[...] + 1e-5)
        t1 = be_r[...] - rm_r[...] * s1
        xv = x_r[...]
        ta_r[0:N, 0:HID] = _dotg(xv, wl_r[...]) * s1
        ta_r[0:N, HID:HID + 1] = jnp.ones((N, 1), jnp.float32)
        ta_r[0:N, HID + 1:WA] = jnp.zeros((N, WA - HID - 1), jnp.float32)
        ta_r[N:NP, :] = jnp.zeros((NP - N, WA), jnp.float32)
        xr_r[...] = _dotg(xv, wr_r[...]) * s1 + (bl_r[...] * s1 + t1)
        idn_r[...] = _dotg(xv, ws_r[...]) + bs_r[...]

    return pl.pallas_call(
        body,
        out_shape=(
            jax.ShapeDtypeStruct((NP, WA), jnp.float32),
            jax.ShapeDtypeStruct((N, HID), jnp.float32),
            jax.ShapeDtypeStruct((N, HID2), jnp.float32),
        ),
    )(x, W_l1, b_l1, W_r1, gamma1, beta1, rm1, rv1, Ws, bs)


def _tc2(P, xr, Wg, a_src, a_dst):
    def body(p_r, xr_r, wg_r, as_r, ad_r, ts_r, td_r, cnt_r):
        S = p_r[0][0:N] + p_r[1][0:N]
        cnt = jnp.maximum(S[:, HID:HID + 1], 1.0)
        h1 = jnp.maximum(S[:, 0:HID] / cnt + xr_r[...], 0.0)
        wg = wg_r[...]
        def _proj(avec, kk):
            return lax.dot_general(avec, wg[kk * HID:(kk + 1) * HID, :],
                                   (((1,), (0,)), ((), ())),
                                   preferred_element_type=jnp.float32)
        rows = []
        for kk in range(HEADS):
            rows.append(_proj(as_r[kk:kk + 1, :], kk))
        for kk in range(HEADS):
            rows.append(_proj(ad_r[kk:kk + 1, :], kk))
        M = jnp.concatenate(rows, axis=0)
        aa = _dotg(h1, M)
        als = aa[:, 0:HEADS]
        ald = aa[:, HEADS:2 * HEADS]
        gmax = jnp.max(als, axis=0, keepdims=True)
        z = gmax + ald
        cc = jnp.where(z > 0.0, z, 0.2 * z)
        ts_r[0:N, 0:HID] = h1
        ts_r[0:N, HID:HID + HEADS] = als
        ts_r[0:N, HID + HEADS:WB_S] = jnp.zeros((N, WB_S - HID - HEADS), jnp.float32)
        ts_r[N:NP, :] = jnp.zeros((NP - N, WB_S), jnp.float32)
        td_r[0:N, 0:HEADS] = ald
        td_r[0:N, HEADS:2 * HEADS] = cc
        td_r[0:N, 2 * HEADS:WB_D] = jnp.zeros((N, WB_D - 2 * HEADS), jnp.float32)
        td_r[N:NP, :] = jnp.zeros((NP - N, WB_D), jnp.float32)
        cnt_r[...] = jnp.broadcast_to(cnt, (N, 8))

    return pl.pallas_call(
        body,
        out_shape=(
            jax.ShapeDtypeStruct((NP, WB_S), jnp.float32),
            jax.ShapeDtypeStruct((NP, WB_D), jnp.float32),
            jax.ShapeDtypeStruct((N, 8), jnp.float32),
        ),
    )(P, xr, Wg, a_src, a_dst)


def _tc3(P2, Wg, bg, gamma2, beta2, rm2, rv2,
         W_l3, b_l3, W_r3, gamma3, beta3, rm3, rv3):
    def body(p2_r, wg_r, bg_r, g2_r, b2_r, rm2_r, rv2_r,
             wl3_r, bl3_r, wr3_r, g3_r, b3_r, rm3_r, rv3_r, tb_r, h2r_r):
        wg = wg_r[...]
        gat = jnp.zeros((N, HID), jnp.float32)
        for kk in range(HEADS):
            ci, slot = kk // 2, kk % 2
            numer = p2_r[ci, slot][0:N, 0:HID]
            den = p2_r[ci, slot][0:N, HID:HID + 1]
            ok = _dotg(numer, wg[kk * HID:(kk + 1) * HID, :])
            gat = gat + jnp.where(den > 0.0, ok / jnp.maximum(den, 1e-30), 0.0)
        gat = gat * 0.25 + bg_r[...]
        s2 = g2_r[...] * lax.rsqrt(rv2_r[...] + 1e-5)
        t2 = b2_r[...] - rm2_r[...] * s2
        h2 = jnp.maximum(gat * s2 + t2, 0.0)
        s3 = g3_r[...] * lax.rsqrt(rv3_r[...] + 1e-5)
        t3 = b3_r[...] - rm3_r[...] * s3
        tb_r[0:N, :] = _dotg(h2, wl3_r[...]) * s3
        tb_r[N:NP, :] = jnp.zeros((NP - N, WC), jnp.float32)
        h2r_r[...] = _dotg(h2, wr3_r[...]) * s3 + (bl3_r[...] * s3 + t3)

    return pl.pallas_call(
        body,
        out_shape=(
            jax.ShapeDtypeStruct((NP, WC), jnp.float32),
            jax.ShapeDtypeStruct((N, HID2), jnp.float32),
        ),
    )(P2, Wg, bg, gamma2, beta2, rm2, rv2,
      W_l3, b_l3, W_r3, gamma3, beta3, rm3, rv3)


def _tc4(P3, cntv, h2r, idn, Wc1, bc1, Wc2, bc2):
    def body(p3_r, cnt_r, h2r_r, idn_r, wc1_r, bc1_r, wc2_r, bc2_r, out_r):
        S3 = p3_r[0][0:N] + p3_r[1][0:N]
        cnt = cnt_r[:, 0:1]
        h3 = jnp.maximum(S3 / cnt + h2r_r[...], 0.0)
        emb = h3 + idn_r[...]
        z = jnp.maximum(_dotg(emb, wc1_r[...]) + bc1_r[...], 0.0)
        l = _dotg(z, wc2_r[...]) + bc2_r[...]
        m = jnp.max(l, axis=1, keepdims=True)
        lse = m + jnp.log(jnp.sum(jnp.exp(l - m), axis=1, keepdims=True))
        out_r[...] = l - lse

    return pl.pallas_call(
        body,
        out_shape=jax.ShapeDtypeStruct((N, 2), jnp.float32),
    )(P3, cntv, h2r, idn, Wc1, bc1, Wc2, bc2)


def kernel(x, edge_index, W_l1, b_l1, W_r1, gamma1, beta1, rm1, rv1,
           Wg, a_src, a_dst, bg, gamma2, beta2, rm2, rv2,
           W_l3, b_l3, W_r3, gamma3, beta3, rm3, rv3,
           Ws, bs, Wc1, bc1, Wc2, bc2):
    pad_row = N + (jnp.arange(PE - E, dtype=jnp.int32) % (NP - N))
    src2d = jnp.concatenate([edge_index[0], pad_row]).reshape(PCHUNK, CH)
    dst2d = jnp.concatenate([edge_index[1], pad_row]).reshape(PCHUNK, CH)
    r = lambda v: v.reshape(1, -1)

    tA, xr, idn = _tc1(x, W_l1, r(b_l1), W_r1, r(gamma1), r(beta1),
                       r(rm1), r(rv1), Ws, r(bs))
    P = _seg_sum(tA, src2d, dst2d, jnp.zeros((NP, WA), jnp.float32), WA)
    tS, tD, cntv = _tc2(P, xr, Wg, a_src, a_dst)
    P2 = _gat_edge(tS, tD, src2d, dst2d, jnp.zeros((NP, WB_O), jnp.float32))
    tB, h2r = _tc3(P2, Wg, r(bg), r(gamma2), r(beta2), r(rm2), r(rv2),
                   W_l3, r(b_l3), W_r3, r(gamma3), r(beta3), r(rm3), r(rv3))
    P3 = _seg_sum(tB, src2d, dst2d, jnp.zeros((NP, WC), jnp.float32), WC)
    return _tc4(P3, cntv, h2r, idn, Wc1, r(bc1), Wc2, r(bc2))


# speedup vs baseline: 1.7047x; 1.5636x over previous
import functools

import jax
import jax.numpy as jnp
from jax import lax
from jax.experimental import pallas as pl
from jax.experimental.pallas import tpu as pltpu
from jax.experimental.pallas import tpu_sc as plsc

N = 10000
E = 320000
IN = 128
HID = 64
HID2 = 32
HEADS = 4

NC, NS = 2, 16
NW = NC * NS
CH = 128
NP = 10240
PE = 327680
PCHUNK = PE // CH
FULL = PCHUNK // NW
RPS = NP // NS

WA = 80
WB_S = 80
WB_D = 16
WB_O = 72
WC = 32


def _mesh():
    return plsc.VectorSubcoreMesh(core_axis_name="c", subcore_axis_name="s")


def _stage_indices(src_h, dst_h, src_v, dst_v, w):
    pltpu.sync_copy(src_h.at[pl.ds(w * FULL, FULL)], src_v)
    pltpu.sync_copy(dst_h.at[pl.ds(w * FULL, FULL)], dst_v)


def _seg_sum(table, src2d, dst2d, zeros, width):
    @functools.partial(
        pl.kernel,
        out_type=jax.ShapeDtypeStruct((NC, NP, width), jnp.float32),
        mesh=_mesh(),
        compiler_params=pltpu.CompilerParams(use_tc_tiling_on_sc=False, needs_layout_passes=False),
        scratch_types=[
            pltpu.VMEM((FULL, CH), jnp.int32),
            pltpu.VMEM((FULL, CH), jnp.int32),
            pltpu.VMEM((4, CH, width), jnp.float32),
            pltpu.VMEM_SHARED((NP, width), jnp.float32),
            [pltpu.SemaphoreType.DMA] * 4,
        ],
    )
    def k(table_h, src_h, dst_h, zeros_h, out_h, src_v, dst_v, rows_v, acc,
          gsems):
        cid = lax.axis_index("c")
        sid = lax.axis_index("s")
        w = sid * NC + cid
        pltpu.sync_copy(zeros_h.at[pl.ds(sid * RPS, RPS)],
                        acc.at[pl.ds(sid * RPS, RPS)])
        _stage_indices(src_h, dst_h, src_v, dst_v, w)
        plsc.subcore_barrier()

        for b in range(4):
            pltpu.async_copy(table_h.at[src_v.at[b]], rows_v.at[b], gsems[b])

        def gbody(q, carry):
            for b in range(4):
                j = q * 4 + b
                pltpu.make_async_copy(table_h.at[src_v.at[j]],
                                      rows_v.at[b], gsems[b]).wait()
                pltpu.sync_copy(rows_v.at[b], acc.at[dst_v.at[j]], add=True)
                nj = j + 4

                @pl.when(nj < FULL)
                def _():
                    pltpu.async_copy(table_h.at[src_v.at[nj]],
                                     rows_v.at[b], gsems[b])
            return carry

        lax.fori_loop(0, FULL // 4, gbody, 0)
        plsc.subcore_barrier()
        pltpu.sync_copy(acc.at[pl.ds(sid * RPS, RPS)],
                        out_h.at[cid, pl.ds(sid * RPS, RPS)])

    return k(table, src2d, dst2d, zeros)


def _gat_edge(t_src, t_dst, src2d, dst2d, zeros):
    @functools.partial(
        pl.kernel,
        out_type=jax.ShapeDtypeStruct((NC, 2, NP, WB_O), jnp.float32),
        mesh=_mesh(),
        compiler_params=pltpu.CompilerParams(use_tc_tiling_on_sc=False, needs_layout_passes=False),
        scratch_types=[
            pltpu.VMEM((FULL, CH), jnp.int32),
            pltpu.VMEM((FULL, CH), jnp.int32),
            pltpu.VMEM((2, CH, WB_S), jnp.float32),
            pltpu.VMEM((2, CH, WB_D), jnp.float32),
            pltpu.VMEM((2, CH, WB_O), jnp.float32),
            pltpu.VMEM_SHARED((NP, WB_O), jnp.float32),
            [pltpu.SemaphoreType.DMA] * 2,
            [pltpu.SemaphoreType.DMA] * 2,
            [pltpu.SemaphoreType.DMA] * 2,
        ],
    )
    def k(ts_h, td_h, src_h, dst_h, zeros_h, out_h,
          src_v, dst_v, srows, drows, orows, acc, sem_s, sem_d, sem_o):
        cid = lax.axis_index("c")
        sid = lax.axis_index("s")
        w = sid * NC + cid
        _stage_indices(src_h, dst_h, src_v, dst_v, w)

        zero16 = jnp.zeros((16,), jnp.float32)

        def zbody(rz, carry):
            orows[0, rz, pl.ds(HID - 8, 16)] = zero16
            orows[1, rz, pl.ds(HID - 8, 16)] = zero16
            return carry

        lax.fori_loop(0, CH, zbody, 0)

        for p in range(2):
            pltpu.sync_copy(zeros_h.at[pl.ds(sid * RPS, RPS)],
                            acc.at[pl.ds(sid * RPS, RPS)])
            plsc.subcore_barrier()
            cals = 64 + 2 * cid + p
            cald = 2 * cid + p

            for b in range(2):
                pltpu.async_copy(ts_h.at[src_v.at[b]], srows.at[b], sem_s[b])
                pltpu.async_copy(td_h.at[dst_v.at[b]], drows.at[b], sem_d[b])

            def pbody(q, carry):
                for b in range(2):
                    j = q * 2 + b
                    o = b
                    pltpu.make_async_copy(ts_h.at[src_v.at[j]],
                                          srows.at[b], sem_s[b]).wait()
                    pltpu.make_async_copy(td_h.at[dst_v.at[j]],
                                          drows.at[b], sem_d[b]).wait()

                    @pl.when(j >= 2)
                    def _():
                        pltpu.make_async_copy(orows.at[o],
                                              acc.at[dst_v.at[j]],
                                              sem_o[o]).wait()

                    for g in range(CH // 16):
                        ie = g * 16 + lax.iota(jnp.int32, 16)
                        als = plsc.load_gather(
                            srows.at[b], [ie, jnp.full((16,), cals, jnp.int32)])
                        ald = plsc.load_gather(
                            drows.at[b], [ie, jnp.full((16,), cald, jnp.int32)])
                        cc = plsc.load_gather(
                            drows.at[b],
                            [ie, jnp.full((16,), 4 + cald, jnp.int32)])
                        e = als + ald
                        e = jnp.where(e > 0.0, e, 0.2 * e)
                        ex = jnp.exp(e - cc)
                        plsc.store_scatter(
                            orows.at[o], [ie, jnp.full((16,), HID, jnp.int32)],
                            ex)
                        for jj in range(16):
                            rr = g * 16 + jj
                            eb = jnp.full((16,), ex[jj], jnp.float32)
                            for kk in range(4):
                                v = srows[b, rr, pl.ds(kk * 16, 16)]
                                orows[o, rr, pl.ds(kk * 16, 16)] = v * eb
                    pltpu.async_copy(orows.at[o], acc.at[dst_v.at[j]],
                                     sem_o[o], add=True)
                    nj = j + 2

                    @pl.when(nj < FULL)
                    def _():
                        pltpu.async_copy(ts_h.at[src_v.at[nj]],
                                         srows.at[b], sem_s[b])
                        pltpu.async_copy(td_h.at[dst_v.at[nj]],
                                         drows.at[b], sem_d[b])
                return carry

            lax.fori_loop(0, FULL // 2, pbody, 0)
            for o in range(2):
                pltpu.make_async_copy(orows.at[o], acc.at[dst_v.at[0]],
                                      sem_o[o]).wait()
            plsc.subcore_barrier()
            pltpu.sync_copy(acc.at[pl.ds(sid * RPS, RPS)],
                            out_h.at[cid, p, pl.ds(sid * RPS, RPS)])

    return k(t_src, t_dst, src2d, dst2d, zeros)


def _dotg(a, b):
    return lax.dot_general(a, b, (((1,), (1,)), ((), ())),
                           preferred_element_type=jnp.float32)


def _tc1(x, W_l1, b_l1, W_r1, gamma1, beta1, rm1, rv1, Ws, bs):
    def body(x_r, wl_r, bl_r, wr_r, g_r, be_r, rm_r, rv_r, ws_r, bs_r,
             ta_r, xr_r, idn_r):
        s1 = g_r[...] * lax.rsqrt(rv_r[...] + 1e-5)
        t1 = be_r[...] - rm_r[...] * s1
        xv = x_r[...]
        ta_r[0:N, 0:HID] = _dotg(xv, wl_r[...]) * s1
        ta_r[0:N, HID:HID + 1] = jnp.ones((N, 1), jnp.float32)
        ta_r[0:N, HID + 1:WA] = jnp.zeros((N, WA - HID - 1), jnp.float32)
        ta_r[N:NP, :] = jnp.zeros((NP - N, WA), jnp.float32)
        xr_r[...] = _dotg(xv, wr_r[...]) * s1 + (bl_r[...] * s1 + t1)
        idn_r[...] = _dotg(xv, ws_r[...]) + bs_r[...]

    return pl.pallas_call(
        body,
        out_shape=(
            jax.ShapeDtypeStruct((NP, WA), jnp.float32),
            jax.ShapeDtypeStruct((N, HID), jnp.float32),
            jax.ShapeDtypeStruct((N, HID2), jnp.float32),
        ),
    )(x, W_l1, b_l1, W_r1, gamma1, beta1, rm1, rv1, Ws, bs)


def _tc2(P, xr, Wg, a_src, a_dst):
    def body(p_r, xr_r, wg_r, as_r, ad_r, ts_r, td_r, cnt_r):
        S = p_r[0][0:N] + p_r[1][0:N]
        cnt = jnp.maximum(S[:, HID:HID + 1], 1.0)
        h1 = jnp.maximum(S[:, 0:HID] / cnt + xr_r[...], 0.0)
        wg = wg_r[...]
        def _proj(avec, kk):
            return lax.dot_general(avec, wg[kk * HID:(kk + 1) * HID, :],
                                   (((1,), (0,)), ((), ())),
                                   preferred_element_type=jnp.float32)
        rows = []
        for kk in range(HEADS):
            rows.append(_proj(as_r[kk:kk + 1, :], kk))
        for kk in range(HEADS):
            rows.append(_proj(ad_r[kk:kk + 1, :], kk))
        M = jnp.concatenate(rows, axis=0)
        aa = _dotg(h1, M)
        als = aa[:, 0:HEADS]
        ald = aa[:, HEADS:2 * HEADS]
        gmax = jnp.max(als, axis=0, keepdims=True)
        z = gmax + ald
        cc = jnp.where(z > 0.0, z, 0.2 * z)
        ts_r[0:N, 0:HID] = h1
        ts_r[0:N, HID:HID + HEADS] = als
        ts_r[0:N, HID + HEADS:WB_S] = jnp.zeros((N, WB_S - HID - HEADS), jnp.float32)
        ts_r[N:NP, :] = jnp.zeros((NP - N, WB_S), jnp.float32)
        td_r[0:N, 0:HEADS] = ald
        td_r[0:N, HEADS:2 * HEADS] = cc
        td_r[0:N, 2 * HEADS:WB_D] = jnp.zeros((N, WB_D - 2 * HEADS), jnp.float32)
        td_r[N:NP, :] = jnp.zeros((NP - N, WB_D), jnp.float32)
        cnt_r[...] = jnp.broadcast_to(cnt, (N, 8))

    return pl.pallas_call(
        body,
        out_shape=(
            jax.ShapeDtypeStruct((NP, WB_S), jnp.float32),
            jax.ShapeDtypeStruct((NP, WB_D), jnp.float32),
            jax.ShapeDtypeStruct((N, 8), jnp.float32),
        ),
    )(P, xr, Wg, a_src, a_dst)


def _tc3(P2, Wg, bg, gamma2, beta2, rm2, rv2,
         W_l3, b_l3, W_r3, gamma3, beta3, rm3, rv3):
    def body(p2_r, wg_r, bg_r, g2_r, b2_r, rm2_r, rv2_r,
             wl3_r, bl3_r, wr3_r, g3_r, b3_r, rm3_r, rv3_r, tb_r, h2r_r):
        wg = wg_r[...]
        gat = jnp.zeros((N, HID), jnp.float32)
        for kk in range(HEADS):
            ci, slot = kk // 2, kk % 2
            numer = p2_r[ci, slot][0:N, 0:HID]
            den = p2_r[ci, slot][0:N, HID:HID + 1]
            ok = _dotg(numer, wg[kk * HID:(kk + 1) * HID, :])
            gat = gat + jnp.where(den > 0.0, ok / jnp.maximum(den, 1e-30), 0.0)
        gat = gat * 0.25 + bg_r[...]
        s2 = g2_r[...] * lax.rsqrt(rv2_r[...] + 1e-5)
        t2 = b2_r[...] - rm2_r[...] * s2
        h2 = jnp.maximum(gat * s2 + t2, 0.0)
        s3 = g3_r[...] * lax.rsqrt(rv3_r[...] + 1e-5)
        t3 = b3_r[...] - rm3_r[...] * s3
        tb_r[0:N, :] = _dotg(h2, wl3_r[...]) * s3
        tb_r[N:NP, :] = jnp.zeros((NP - N, WC), jnp.float32)
        h2r_r[...] = _dotg(h2, wr3_r[...]) * s3 + (bl3_r[...] * s3 + t3)

    return pl.pallas_call(
        body,
        out_shape=(
            jax.ShapeDtypeStruct((NP, WC), jnp.float32),
            jax.ShapeDtypeStruct((N, HID2), jnp.float32),
        ),
    )(P2, Wg, bg, gamma2, beta2, rm2, rv2,
      W_l3, b_l3, W_r3, gamma3, beta3, rm3, rv3)


def _tc4(P3, cntv, h2r, idn, Wc1, bc1, Wc2, bc2):
    def body(p3_r, cnt_r, h2r_r, idn_r, wc1_r, bc1_r, wc2_r, bc2_r, out_r):
        S3 = p3_r[0][0:N] + p3_r[1][0:N]
        cnt = cnt_r[:, 0:1]
        h3 = jnp.maximum(S3 / cnt + h2r_r[...], 0.0)
        emb = h3 + idn_r[...]
        z = jnp.maximum(_dotg(emb, wc1_r[...]) + bc1_r[...], 0.0)
        l = _dotg(z, wc2_r[...]) + bc2_r[...]
        m = jnp.max(l, axis=1, keepdims=True)
        lse = m + jnp.log(jnp.sum(jnp.exp(l - m), axis=1, keepdims=True))
        out_r[...] = l - lse

    return pl.pallas_call(
        body,
        out_shape=jax.ShapeDtypeStruct((N, 2), jnp.float32),
    )(P3, cntv, h2r, idn, Wc1, bc1, Wc2, bc2)


def kernel(x, edge_index, W_l1, b_l1, W_r1, gamma1, beta1, rm1, rv1,
           Wg, a_src, a_dst, bg, gamma2, beta2, rm2, rv2,
           W_l3, b_l3, W_r3, gamma3, beta3, rm3, rv3,
           Ws, bs, Wc1, bc1, Wc2, bc2):
    pad_row = N + (jnp.arange(PE - E, dtype=jnp.int32) % (NP - N))
    src2d = jnp.concatenate([edge_index[0], pad_row]).reshape(PCHUNK, CH)
    dst2d = jnp.concatenate([edge_index[1], pad_row]).reshape(PCHUNK, CH)
    r = lambda v: v.reshape(1, -1)

    tA, xr, idn = _tc1(x, W_l1, r(b_l1), W_r1, r(gamma1), r(beta1),
                       r(rm1), r(rv1), Ws, r(bs))
    P = _seg_sum(tA, src2d, dst2d, jnp.zeros((NP, WA), jnp.float32), WA)
    tS, tD, cntv = _tc2(P, xr, Wg, a_src, a_dst)
    P2 = _gat_edge(tS, tD, src2d, dst2d, jnp.zeros((NP, WB_O), jnp.float32))
    tB, h2r = _tc3(P2, Wg, r(bg), r(gamma2), r(beta2), r(rm2), r(rv2),
                   W_l3, r(b_l3), W_r3, r(gamma3), r(beta3), r(rm3), r(rv3))
    P3 = _seg_sum(tB, src2d, dst2d, jnp.zeros((NP, WC), jnp.float32), WC)
    return _tc4(P3, cntv, h2r, idn, Wc1, r(bc1), Wc2, r(bc2))
